# Initial kernel scaffold; baseline (speedup 1.0000x reference)
import jax
import jax.numpy as jnp
from jax.experimental import pallas as pl


def kernel(x, edge_index, W, a_src, a_dst):
    raise NotImplementedError("write your pallas kernel here")


# speedup vs baseline: 20.0614x; 20.0614x over previous
import functools

import jax
import jax.numpy as jnp
from jax import lax
from jax.experimental import pallas as pl
from jax.experimental.pallas import tpu as pltpu
from jax.experimental.pallas import tpu_sc as plsc

N = 10000
E = 320000
D = 128

NC = 2
NS = 16
NW = NC * NS
EPW = E // NW
CK = 80
NCH = EPW // CK
NSG = 5
SG = NCH // NSG
STRIPE = 624
ZR = 16


def _k1_body(x_ref, w_ref, asrc_ref, adst_ref, h_ref, s_ref, d_ref, m_ref):
    i = pl.program_id(0)
    h = jnp.dot(x_ref[...], w_ref[...], preferred_element_type=jnp.float32)
    h_ref[...] = h
    s = jnp.dot(h, asrc_ref[...], preferred_element_type=jnp.float32)
    d = jnp.dot(h, adst_ref[...], preferred_element_type=jnp.float32)
    s_ref[...] = s[:, None]
    d_ref[...] = d[:, None]
    bs = jnp.max(s)
    bd = jnp.max(d)

    @pl.when(i == 0)
    def _init():
        m_ref[0, 0] = bs
        m_ref[0, 1] = bd

    @pl.when(i > 0)
    def _acc():
        m_ref[0, 0] = jnp.maximum(m_ref[0, 0], bs)
        m_ref[0, 1] = jnp.maximum(m_ref[0, 1], bd)


_K1_BLK = 1000


def _k1(x, W, a_src, a_dst):
    return pl.pallas_call(
        _k1_body,
        grid=(N // _K1_BLK,),
        in_specs=[
            pl.BlockSpec((_K1_BLK, D), lambda i: (i, 0)),
            pl.BlockSpec((D, D), lambda i: (0, 0)),
            pl.BlockSpec((D,), lambda i: (0,)),
            pl.BlockSpec((D,), lambda i: (0,)),
        ],
        out_specs=[
            pl.BlockSpec((_K1_BLK, D), lambda i: (i, 0)),
            pl.BlockSpec((_K1_BLK, 1), lambda i: (i, 0)),
            pl.BlockSpec((_K1_BLK, 1), lambda i: (i, 0)),
            pl.BlockSpec((1, 16), lambda i: (0, 0), memory_space=pltpu.SMEM),
        ],
        out_shape=[
            jax.ShapeDtypeStruct((N, D), jnp.float32),
            jax.ShapeDtypeStruct((N, 1), jnp.float32),
            jax.ShapeDtypeStruct((N, 1), jnp.float32),
            jax.ShapeDtypeStruct((1, 16), jnp.float32),
        ],
    )(x, W, a_src, a_dst)


def _k2_body(src_hbm, dst_hbm, s_hbm, d_hbm, m_hbm, h_hbm,
             den_hbm, parts_hbm,
             src_v, dst_v, s_v, d_v, den_v, p_v, rows_v, m_v, acc):
    cid = lax.axis_index("c")
    sid = lax.axis_index("s")
    wid = sid * NC + cid

    pltpu.sync_copy(s_hbm, s_v)
    pltpu.sync_copy(d_hbm, d_v)
    pltpu.sync_copy(m_hbm, m_v)
    mvec = m_v[0, pl.ds(0, 16)]
    c_sh = jnp.maximum(mvec[0] + mvec[1], 0.0)

    zeros16 = jnp.zeros((16,), jnp.float32)

    def _zden(i, carry):
        den_v[0, pl.ds(i * 16, 16)] = zeros16
        return carry
    lax.fori_loop(0, N // 16, _zden, 0)

    def _zbuf(i, carry):
        for t in range(8):
            rows_v[i, pl.ds(t * 16, 16)] = zeros16
        return carry
    lax.fori_loop(0, ZR, _zbuf, 0)

    base = pl.multiple_of(sid * STRIPE, 8)
    zsrc = rows_v.at[pl.ds(0, ZR)]

    def _zacc(r, carry):
        off = pl.multiple_of(base + r * ZR, 8)
        pltpu.sync_copy(zsrc, acc.at[pl.ds(off, ZR)])
        return carry
    lax.fori_loop(0, STRIPE // ZR, _zacc, 0)

    @pl.when(sid == NS - 1)
    def _ztail():
        pltpu.sync_copy(zsrc, acc.at[pl.ds(NS * STRIPE, N - NS * STRIPE)])

    plsc.subcore_barrier()

    def _chunk(ci, carry):
        for u in range(CK // 16):
            sl = pl.ds(u * 16, 16)
            si = src_v[ci, sl]
            di = dst_v[ci, sl]
            z = plsc.load_gather(s_v, [si]) + plsc.load_gather(d_v, [di])
            e = jnp.where(z > 0, z, z * jnp.float32(0.2))
            p = jnp.exp(e - c_sh)
            p_v[sl] = p
            plsc.addupdate_scatter(den_v.at[0], [di], p)

        pltpu.sync_copy(h_hbm.at[src_v.at[ci]], rows_v)

        def _scale(j, carry2):
            pj = plsc.load_gather(p_v, [jnp.full((16,), j, jnp.int32)])
            for t in range(8):
                sl = pl.ds(t * 16, 16)
                rows_v[j, sl] = rows_v[j, sl] * pj
            return carry2
        lax.fori_loop(0, CK, _scale, 0)

        pltpu.sync_copy(rows_v, acc.at[dst_v.at[ci]], add=True)
        return carry

    for g in range(NSG):
        pltpu.sync_copy(src_hbm.at[wid, g], src_v)
        pltpu.sync_copy(dst_hbm.at[wid, g], dst_v)
        lax.fori_loop(0, SG, _chunk, 0)

    pltpu.sync_copy(den_v, den_hbm.at[wid])

    plsc.subcore_barrier()
    pltpu.sync_copy(acc.at[pl.ds(base, STRIPE)],
                    parts_hbm.at[cid, pl.ds(base, STRIPE)])

    @pl.when(sid == NS - 1)
    def _ctail():
        pltpu.sync_copy(acc.at[pl.ds(NS * STRIPE, N - NS * STRIPE)],
                        parts_hbm.at[cid, pl.ds(NS * STRIPE, N - NS * STRIPE)])


def _k2(src3, dst3, s, d, m, h):
    mesh = plsc.VectorSubcoreMesh(core_axis_name="c", subcore_axis_name="s")
    f = pl.kernel(
        _k2_body,
        out_type=[
            jax.ShapeDtypeStruct((NW, 1, N), jnp.float32),
            jax.ShapeDtypeStruct((NC, N, D), jnp.float32),
        ],
        mesh=mesh,
        scratch_types=[
            pltpu.VMEM((SG, CK), jnp.int32),
            pltpu.VMEM((SG, CK), jnp.int32),
            pltpu.VMEM((N,), jnp.float32),
            pltpu.VMEM((N,), jnp.float32),
            pltpu.VMEM((1, N), jnp.float32),
            pltpu.VMEM((CK,), jnp.float32),
            pltpu.VMEM((CK, D), jnp.float32),
            pltpu.VMEM((1, 16), jnp.float32),
            pltpu.VMEM_SHARED((N, D), jnp.float32),
        ],
        compiler_params=pltpu.CompilerParams(needs_layout_passes=False),
    )
    return f(src3, dst3, s, d, m, h)


def _k3_body(den_ref, parts_ref, out_ref):
    dsum = jnp.sum(den_ref[:, 0, :], axis=0)
    dinv = 1.0 / (dsum + jnp.float32(1e-16))
    out_ref[...] = (parts_ref[0] + parts_ref[1]) * dinv[:, None]


def _k3(den, parts):
    return pl.pallas_call(
        _k3_body,
        out_shape=jax.ShapeDtypeStruct((N, D), jnp.float32),
    )(den, parts)


@jax.jit
def kernel(x, edge_index, W, a_src, a_dst):
    src3 = edge_index[0].reshape(NW, NSG, SG, CK)
    dst3 = edge_index[1].reshape(NW, NSG, SG, CK)
    h, s, d, m = _k1(x, W, a_src, a_dst)
    den, parts = _k2(src3, dst3, s.reshape(N), d.reshape(N), m, h)
    return _k3(den, parts)
